# Initial kernel scaffold; baseline (speedup 1.0000x reference)
import jax
import jax.numpy as jnp
from jax.experimental import pallas as pl


def kernel(text_input, table, W, b):
    raise NotImplementedError("write your pallas kernel here")


# speedup vs baseline: 1.8861x; 1.8861x over previous
import functools

import jax
import jax.numpy as jnp
from jax import lax
from jax.experimental import pallas as pl
from jax.experimental.pallas import tpu as pltpu
from jax.experimental.pallas import tpu_sc as plsc

VOCAB = 1000000
D = 32
IMG = 2048
B = 4096
L = 200

NC = 2
NS = 16
NW = NC * NS
RPW = B // NW
C0, C1 = 128, 72
SCALE = 1.0 / VOCAB
LANES = 16


def _pool_body(ti_hbm, table_hbm, out_hbm, idx_v, rows_v, acc_v, sem):
    wid = lax.axis_index("s") * NC + lax.axis_index("c")
    base = wid * RPW
    pltpu.sync_copy(ti_hbm.at[pl.ds(base, RPW)], idx_v)

    def row_body(r, _):
        cp0 = pltpu.async_copy(table_hbm.at[idx_v.at[r, pl.ds(0, C0)]],
                               rows_v.at[pl.ds(0, C0)], sem)
        cp1 = pltpu.async_copy(table_hbm.at[idx_v.at[r, pl.ds(C0, C1)]],
                               rows_v.at[pl.ds(C0, C1)], sem)
        cp0.wait()
        cp1.wait()

        def acc_body(j, carry):
            a0, a1 = carry
            return (a0 + rows_v[j, pl.ds(0, LANES)],
                    a1 + rows_v[j, pl.ds(LANES, LANES)])

        z = jnp.zeros((LANES,), jnp.float32)
        a0, a1 = lax.fori_loop(0, L, acc_body, (z, z))
        acc_v[r, pl.ds(0, LANES)] = a0 * SCALE
        acc_v[r, pl.ds(LANES, LANES)] = a1 * SCALE
        return 0

    lax.fori_loop(0, RPW, row_body, 0)
    pltpu.sync_copy(acc_v, out_hbm.at[pl.ds(base, RPW)])


_pool = pl.kernel(
    _pool_body,
    out_type=jax.ShapeDtypeStruct((B, D), jnp.float32),
    mesh=plsc.VectorSubcoreMesh(core_axis_name="c", subcore_axis_name="s",
                                num_cores=NC, num_subcores=NS),
    scratch_types=[
        pltpu.VMEM((RPW, L), jnp.int32),
        pltpu.VMEM((L, D), jnp.float32),
        pltpu.VMEM((RPW, D), jnp.float32),
        pltpu.SemaphoreType.DMA,
    ],
    compiler_params=pltpu.CompilerParams(use_tc_tiling_on_sc=False),
)

BM = 512


def _mm_body(x_ref, w_ref, b_ref, o_ref):
    o_ref[...] = lax.dot_general(
        x_ref[...], w_ref[...], (((1,), (1,)), ((), ())),
        preferred_element_type=jnp.float32) + b_ref[...]


_matmul = pl.pallas_call(
    _mm_body,
    grid=(B // BM,),
    in_specs=[
        pl.BlockSpec((BM, D), lambda i: (i, 0)),
        pl.BlockSpec((IMG, D), lambda i: (0, 0)),
        pl.BlockSpec((1, IMG), lambda i: (0, 0)),
    ],
    out_specs=pl.BlockSpec((BM, IMG), lambda i: (i, 0)),
    out_shape=jax.ShapeDtypeStruct((B, IMG), jnp.float32),
)


def kernel(text_input, table, W, b):
    sum_embeds = _pool(text_input, table)
    return _matmul(sum_embeds, W, b.reshape(1, IMG))


# speedup vs baseline: 5.5191x; 2.9262x over previous
import functools

import jax
import jax.numpy as jnp
from jax import lax
from jax.experimental import pallas as pl
from jax.experimental.pallas import tpu as pltpu
from jax.experimental.pallas import tpu_sc as plsc

VOCAB = 1000000
D = 32
IMG = 2048
B = 4096
L = 200

NC = 2
NS = 16
NW = NC * NS
RPW = B // NW
C0, C1 = 128, 72
SCALE = 1.0 / VOCAB
LANES = 16


_OFFS = tuple(range(0, L - 15, LANES)) + ((L - LANES),)


NSLOT = 4


def _pool_body(ti_hbm, table_hbm, out_hbm, idx_v, idx_t, rows_v, acc_v,
               sem0, sem1, sem2, sem3):
    wid = lax.axis_index("s") * NC + lax.axis_index("c")
    base = wid * RPW
    sems = (sem0, sem1, sem2, sem3)
    pltpu.sync_copy(ti_hbm.at[pl.ds(base, RPW)], idx_v)

    def fire(r, s):
        for off in _OFFS:
            v = idx_v[r, pl.ds(off, LANES)]
            k = (jnp.bitwise_and(v, -CB)
                 + ((v & (CQ - 1)) << 2) + ((v & (CB - 1)) >> CSH))
            idx_t[s, pl.ds(off, LANES)] = k
        pltpu.async_copy(table_hbm.at[idx_t.at[s, pl.ds(0, C0)]],
                         rows_v.at[s, pl.ds(0, C0)], sems[s])
        pltpu.async_copy(table_hbm.at[idx_t.at[s, pl.ds(C0, C1)]],
                         rows_v.at[s, pl.ds(C0, C1)], sems[s])

    def wait(s):
        pltpu.make_async_copy(table_hbm.at[idx_t.at[s, pl.ds(0, C0)]],
                              rows_v.at[s, pl.ds(0, C0)], sems[s]).wait()
        pltpu.make_async_copy(table_hbm.at[idx_t.at[s, pl.ds(C0, C1)]],
                              rows_v.at[s, pl.ds(C0, C1)], sems[s]).wait()

    def consume(r, s):
        def acc_body(j, carry):
            a00, a01, a10, a11 = carry
            j0 = 2 * j
            return (a00 + rows_v[s, j0, pl.ds(0, LANES)],
                    a01 + rows_v[s, j0, pl.ds(LANES, LANES)],
                    a10 + rows_v[s, j0 + 1, pl.ds(0, LANES)],
                    a11 + rows_v[s, j0 + 1, pl.ds(LANES, LANES)])

        z = jnp.zeros((LANES,), jnp.float32)
        a00, a01, a10, a11 = lax.fori_loop(0, L // 2, acc_body, (z, z, z, z),
                                           unroll=8)
        acc_v[r, pl.ds(0, LANES)] = (a00 + a10) * SCALE
        acc_v[r, pl.ds(LANES, LANES)] = (a01 + a11) * SCALE

    for s in range(NSLOT - 1):
        fire(s, s)

    def quad_body(g, _):
        r0 = NSLOT * g
        for bslot in range(NSLOT):
            r = r0 + bslot
            wait(bslot)
            consume(r, bslot)
            nxt = r + NSLOT - 1

            @pl.when(nxt < RPW)
            def _():
                fire(nxt, (bslot + NSLOT - 1) % NSLOT)

        return 0

    lax.fori_loop(0, RPW // NSLOT, quad_body, 0)
    pltpu.sync_copy(acc_v, out_hbm.at[pl.ds(base, RPW)])


_pool = pl.kernel(
    _pool_body,
    out_type=jax.ShapeDtypeStruct((B, D), jnp.float32),
    mesh=plsc.VectorSubcoreMesh(core_axis_name="c", subcore_axis_name="s",
                                num_cores=NC, num_subcores=NS),
    scratch_types=[
        pltpu.VMEM((RPW, L), jnp.int32),
        pltpu.VMEM((NSLOT, L), jnp.int32),
        pltpu.VMEM((NSLOT, L, D), jnp.float32),
        pltpu.VMEM((RPW, D), jnp.float32),
        pltpu.SemaphoreType.DMA,
        pltpu.SemaphoreType.DMA,
        pltpu.SemaphoreType.DMA,
        pltpu.SemaphoreType.DMA,
    ],
    compiler_params=pltpu.CompilerParams(use_tc_tiling_on_sc=False),
)

CB = 32768
CQ = CB // 4
CSH = CQ.bit_length() - 1
NG = -(-VOCAB // CB)
NROW = NG * CQ


def _tr_body(xt_ref, o_ref):
    x = xt_ref[...]
    lane = lax.broadcasted_iota(jnp.int32, (D, 128), 1)
    row = lax.broadcasted_iota(jnp.int32, (D, 128), 0)
    acc = None
    for g in range(4):
        eg = (lane == row + 32 * g).astype(jnp.float32)
        p = lax.dot_general(x[:, g * CQ:(g + 1) * CQ], eg,
                            (((0,), (0,)), ((), ())),
                            preferred_element_type=jnp.float32)
        acc = p if acc is None else acc + p
    o_ref[...] = acc


_transpose = pl.pallas_call(
    _tr_body,
    grid=(NG,),
    in_specs=[pl.BlockSpec((D, CB), lambda i: (0, i))],
    out_specs=pl.BlockSpec((CQ, 128), lambda i: (i, 0)),
    out_shape=jax.ShapeDtypeStruct((NROW, 128), jnp.float32),
)

BM = 512


def _mm_body(x_ref, w_ref, b_ref, o_ref):
    o_ref[...] = lax.dot_general(
        x_ref[...], w_ref[...], (((1,), (1,)), ((), ())),
        preferred_element_type=jnp.float32) + b_ref[...]


_matmul = pl.pallas_call(
    _mm_body,
    grid=(B // BM,),
    in_specs=[
        pl.BlockSpec((BM, D), lambda i: (i, 0)),
        pl.BlockSpec((IMG, D), lambda i: (0, 0)),
        pl.BlockSpec((1, IMG), lambda i: (0, 0)),
    ],
    out_specs=pl.BlockSpec((BM, IMG), lambda i: (i, 0)),
    out_shape=jax.ShapeDtypeStruct((B, IMG), jnp.float32),
)


def kernel(text_input, table, W, b):
    t_lin = _transpose(table.T).reshape(NROW * 4, D)
    sum_embeds = _pool(text_input, t_lin)
    return _matmul(sum_embeds, W, b.reshape(1, IMG))


# speedup vs baseline: 5.7962x; 1.0502x over previous
import functools

import jax
import jax.numpy as jnp
from jax import lax
from jax.experimental import pallas as pl
from jax.experimental.pallas import tpu as pltpu
from jax.experimental.pallas import tpu_sc as plsc

VOCAB = 1000000
D = 32
IMG = 2048
B = 4096
L = 200

NC = 2
NS = 16
NW = NC * NS
RPW = B // NW
C0, C1 = 104, 96
SCALE = 1.0 / VOCAB
LANES = 16


_OFFS = tuple(range(0, L - 15, LANES)) + ((L - LANES),)


NSLOT = 8


def _pool_body(ti_hbm, table_hbm, out_hbm, idx_v, idx_t, rows_v, acc_v,
               sem0, sem1, sem2, sem3, sem4, sem5, sem6, sem7):
    wid = lax.axis_index("s") * NC + lax.axis_index("c")
    base = wid * RPW
    sems = (sem0, sem1, sem2, sem3, sem4, sem5, sem6, sem7)
    pltpu.sync_copy(ti_hbm.at[pl.ds(base, RPW)], idx_v)

    def fire(r, s):
        for off in _OFFS:
            v = idx_v[r, pl.ds(off, LANES)]
            k = (jnp.bitwise_and(v, -CB)
                 + ((v & (CQ - 1)) << 2) + ((v & (CB - 1)) >> CSH))
            idx_t[s, pl.ds(off, LANES)] = k
        pltpu.async_copy(table_hbm.at[idx_t.at[s, pl.ds(0, C0)]],
                         rows_v.at[s, pl.ds(0, C0)], sems[s])
        pltpu.async_copy(table_hbm.at[idx_t.at[s, pl.ds(C0, C1)]],
                         rows_v.at[s, pl.ds(C0, C1)], sems[s])

    def wait(s):
        pltpu.make_async_copy(table_hbm.at[idx_t.at[s, pl.ds(0, C0)]],
                              rows_v.at[s, pl.ds(0, C0)], sems[s]).wait()
        pltpu.make_async_copy(table_hbm.at[idx_t.at[s, pl.ds(C0, C1)]],
                              rows_v.at[s, pl.ds(C0, C1)], sems[s]).wait()

    def consume(r, s):
        def acc_body(j, carry):
            a00, a01, a10, a11 = carry
            j0 = 2 * j
            return (a00 + rows_v[s, j0, pl.ds(0, LANES)],
                    a01 + rows_v[s, j0, pl.ds(LANES, LANES)],
                    a10 + rows_v[s, j0 + 1, pl.ds(0, LANES)],
                    a11 + rows_v[s, j0 + 1, pl.ds(LANES, LANES)])

        z = jnp.zeros((LANES,), jnp.float32)
        a00, a01, a10, a11 = lax.fori_loop(0, L // 2, acc_body, (z, z, z, z),
                                           unroll=8)
        acc_v[r, pl.ds(0, LANES)] = (a00 + a10) * SCALE
        acc_v[r, pl.ds(LANES, LANES)] = (a01 + a11) * SCALE

    for s in range(NSLOT - 1):
        fire(s, s)

    def quad_body(g, _):
        r0 = NSLOT * g
        for bslot in range(NSLOT):
            r = r0 + bslot
            wait(bslot)
            consume(r, bslot)
            nxt = r + NSLOT - 1

            @pl.when(nxt < RPW)
            def _():
                fire(nxt, (bslot + NSLOT - 1) % NSLOT)

        return 0

    lax.fori_loop(0, RPW // NSLOT, quad_body, 0)
    pltpu.sync_copy(acc_v, out_hbm.at[pl.ds(base, RPW)])


_pool = pl.kernel(
    _pool_body,
    out_type=jax.ShapeDtypeStruct((B, D), jnp.float32),
    mesh=plsc.VectorSubcoreMesh(core_axis_name="c", subcore_axis_name="s",
                                num_cores=NC, num_subcores=NS),
    scratch_types=[
        pltpu.VMEM((RPW, L), jnp.int32),
        pltpu.VMEM((NSLOT, L), jnp.int32),
        pltpu.VMEM((NSLOT, L, D), jnp.float32),
        pltpu.VMEM((RPW, D), jnp.float32),
        pltpu.SemaphoreType.DMA,
        pltpu.SemaphoreType.DMA,
        pltpu.SemaphoreType.DMA,
        pltpu.SemaphoreType.DMA,
        pltpu.SemaphoreType.DMA,
        pltpu.SemaphoreType.DMA,
        pltpu.SemaphoreType.DMA,
        pltpu.SemaphoreType.DMA,
    ],
    compiler_params=pltpu.CompilerParams(use_tc_tiling_on_sc=False),
)

CB = 32768
CQ = CB // 4
CSH = CQ.bit_length() - 1
NG = -(-VOCAB // CB)
NROW = NG * CQ


def _tr_body(xt_ref, o_ref):
    x = xt_ref[...]
    lane = lax.broadcasted_iota(jnp.int32, (D, 128), 1)
    row = lax.broadcasted_iota(jnp.int32, (D, 128), 0)
    acc = None
    for g in range(4):
        eg = (lane == row + 32 * g).astype(jnp.float32)
        p = lax.dot_general(x[:, g * CQ:(g + 1) * CQ], eg,
                            (((0,), (0,)), ((), ())),
                            preferred_element_type=jnp.float32)
        acc = p if acc is None else acc + p
    o_ref[...] = acc


_transpose = pl.pallas_call(
    _tr_body,
    grid=(NG,),
    in_specs=[pl.BlockSpec((D, CB), lambda i: (0, i))],
    out_specs=pl.BlockSpec((CQ, 128), lambda i: (i, 0)),
    out_shape=jax.ShapeDtypeStruct((NROW, 128), jnp.float32),
)

BM = 512


def _mm_body(x_ref, w_ref, b_ref, o_ref):
    o_ref[...] = lax.dot_general(
        x_ref[...], w_ref[...], (((1,), (1,)), ((), ())),
        preferred_element_type=jnp.float32) + b_ref[...]


_matmul = pl.pallas_call(
    _mm_body,
    grid=(B // BM,),
    in_specs=[
        pl.BlockSpec((BM, D), lambda i: (i, 0)),
        pl.BlockSpec((IMG, D), lambda i: (0, 0)),
        pl.BlockSpec((1, IMG), lambda i: (0, 0)),
    ],
    out_specs=pl.BlockSpec((BM, IMG), lambda i: (i, 0)),
    out_shape=jax.ShapeDtypeStruct((B, IMG), jnp.float32),
)


def kernel(text_input, table, W, b):
    t_lin = _transpose(table.T).reshape(NROW * 4, D)
    sum_embeds = _pool(text_input, t_lin)
    return _matmul(sum_embeds, W, b.reshape(1, IMG))
